# baseline (speedup 1.0000x reference)
import jax
import jax.numpy as jnp
from jax.experimental import pallas as pl

_CONF = 0.01
_BLK = 4000


def _infbox_body(pred_ref, dbox_ref, loc_ref, ind_ref):
    pred = pred_ref[0]
    d = dbox_ref[...]
    ind_ref[0] = pred[:, 4:] > _CONF
    p = pred[:, :4]
    ctr = d[:, :2] + 0.1 * p[:, :2] * d[:, 2:]
    half = 0.5 * d[:, 2:] * jnp.exp(0.2 * p[:, 2:])
    loc_ref[0] = jnp.concatenate([ctr - half, ctr + half], axis=1)


def kernel(predicts, dboxes):
    batch, n, c = predicts.shape
    nblk = n // _BLK
    loc, ind = pl.pallas_call(
        _infbox_body,
        grid=(batch, nblk),
        in_specs=[
            pl.BlockSpec((1, _BLK, c), lambda b, j: (b, j, 0)),
            pl.BlockSpec((_BLK, 4), lambda b, j: (j, 0)),
        ],
        out_specs=[
            pl.BlockSpec((1, _BLK, 4), lambda b, j: (b, j, 0)),
            pl.BlockSpec((1, _BLK, c - 4), lambda b, j: (b, j, 0)),
        ],
        out_shape=[
            jax.ShapeDtypeStruct((batch, n, 4), jnp.float32),
            jax.ShapeDtypeStruct((batch, n, c - 4), jnp.bool_),
        ],
    )(predicts, dboxes)
    return (loc, ind)
